# baseline (speedup 1.0000x reference)
import jax
import jax.numpy as jnp
from jax.experimental import pallas as pl

N, A, C = 128, 8732, 81
SCALE_XY = 1.0 / 0.1
SCALE_WH = 1.0 / 0.2

NB = 8
G = N // NB
AB = 4096
JA = (A + AB - 1) // AB
RW = 512


def _phase1_kernel(ploc_ref, plabel_ref, gloc_ref, glabel_ref, dboxes_ref,
                   con_ref, locred_ref):
    j = pl.program_id(1)

    lbl = glabel_ref[...]
    lane = jax.lax.broadcasted_iota(jnp.int32, (NB, AB), 1)
    valid = (j * AB + lane) < A
    posm = (lbl > 0) & valid

    x = plabel_ref[...]
    e = jnp.exp(x)
    s = jnp.sum(e, axis=1)
    logz = jnp.log(s)
    cidx = jax.lax.broadcasted_iota(jnp.int32, (NB, C, AB), 1)
    tl = jnp.sum(jnp.where(cidx == lbl[:, None, :], x, 0.0), axis=1)
    con_ref[...] = logz - tl

    p = ploc_ref[...]
    g = gloc_ref[...]
    db = dboxes_ref[...]
    gxy = SCALE_XY * (g[:, :2] - db[:, :2]) / db[:, 2:]
    gwh = SCALE_WH * jnp.log(g[:, 2:] / db[:, 2:])
    vec = jnp.concatenate([gxy, gwh], axis=1)
    d = p - vec
    ad = jnp.abs(d)
    sl1 = jnp.sum(jnp.where(ad < 1.0, 0.5 * d * d, ad - 0.5), axis=1)
    ll = jnp.where(posm, sl1, 0.0)

    acc = jnp.zeros((NB, RW), jnp.float32)
    for i in range(AB // RW):
        acc = acc + ll[:, i * RW:(i + 1) * RW]

    @pl.when(j == 0)
    def _():
        locred_ref[...] = jnp.zeros_like(locred_ref)

    locred_ref[...] += acc


def _phase2_kernel(con_ref, glabel_ref, locred_ref, out_ref):
    lbl = glabel_ref[...]
    posm = lbl > 0
    con = con_ref[...]

    pos = jnp.sum(jnp.where(posm, 1.0, 0.0), axis=1, keepdims=True)
    conm = jnp.sum(jnp.where(posm, con, 0.0), axis=1, keepdims=True)
    locm = jnp.sum(locred_ref[...], axis=1, keepdims=True)

    cn = jnp.where(posm, 0.0, con)
    ci = jax.lax.bitcast_convert_type(cn, jnp.int32)

    pos_i = pos.astype(jnp.int32)
    k = jnp.minimum(3 * pos_i, A)
    kk = jnp.maximum(k, 1).astype(jnp.float32)

    prefix = jnp.zeros((N, 1), jnp.int32)
    krem = kk
    for b in range(30, -1, -1):
        hi_mask = jnp.int32(-(1 << b))
        cand = prefix | jnp.int32(1 << b)
        cnt = jnp.sum(jnp.where((ci & hi_mask) == cand, 1.0, 0.0),
                      axis=1, keepdims=True)
        take = krem <= cnt
        prefix = jnp.where(take, cand, prefix)
        krem = jnp.where(take, krem, krem - cnt)

    v = jax.lax.bitcast_convert_type(prefix, jnp.float32)
    gt = cn > v
    t_cnt = jnp.sum(jnp.where(gt, 1.0, 0.0), axis=1, keepdims=True)
    ns = jnp.sum(jnp.where(gt, cn, 0.0), axis=1, keepdims=True)
    neg_total = ns + (k.astype(jnp.float32) - t_cnt) * v

    total = locm + conm + neg_total
    contrib = jnp.where(pos > 0, total / jnp.maximum(pos, 1e-6), 0.0)
    out_ref[...] = jnp.sum(contrib, keepdims=True).reshape(1, 1) / N


@jax.jit
def kernel(ploc, plabel, gloc, glabel, dboxes):
    con, locred = pl.pallas_call(
        _phase1_kernel,
        grid=(G, JA),
        in_specs=[
            pl.BlockSpec((NB, 4, AB), lambda n, j: (n, 0, j)),
            pl.BlockSpec((NB, C, AB), lambda n, j: (n, 0, j)),
            pl.BlockSpec((NB, 4, AB), lambda n, j: (n, 0, j)),
            pl.BlockSpec((NB, AB), lambda n, j: (n, j)),
            pl.BlockSpec((1, 4, AB), lambda n, j: (0, 0, j)),
        ],
        out_specs=[
            pl.BlockSpec((NB, AB), lambda n, j: (n, j)),
            pl.BlockSpec((NB, RW), lambda n, j: (n, 0)),
        ],
        out_shape=[
            jax.ShapeDtypeStruct((N, A), jnp.float32),
            jax.ShapeDtypeStruct((N, RW), jnp.float32),
        ],
    )(ploc, plabel, gloc, glabel, dboxes)

    out = pl.pallas_call(
        _phase2_kernel,
        out_shape=jax.ShapeDtypeStruct((1, 1), jnp.float32),
    )(con, glabel, locred)
    return out[0, 0]


# speedup vs baseline: 3.0966x; 3.0966x over previous
import jax
import jax.numpy as jnp
from jax.experimental import pallas as pl

N, A, C = 128, 8732, 81
SCALE_XY = 1.0 / 0.1
SCALE_WH = 1.0 / 0.2

AB = 256
JA = (A + AB - 1) // AB


def _phase1_kernel(plabel_ref, glabel_ref, ploc_ref, gloc_ref, dbx_ref,
                   con_ref, locred_ref):
    j = pl.program_id(0)

    lbl = glabel_ref[...]
    arow = jax.lax.broadcasted_iota(jnp.int32, (AB, N), 0)
    valid = (j * AB + arow) < A
    posm = (lbl > 0) & valid

    x = plabel_ref[...]
    e = jnp.exp(x)
    s = jnp.sum(e, axis=0)
    logz = jnp.log(s)
    cidx = jax.lax.broadcasted_iota(jnp.int32, (C, AB, N), 0)
    tl = jnp.sum(jnp.where(cidx == lbl[None], x, 0.0), axis=0)
    con_ref[...] = logz - tl

    p = ploc_ref[...]
    g = gloc_ref[...]
    db = dbx_ref[...]
    gxy = SCALE_XY * (g[:2] - db[:2]) / db[2:]
    gwh = SCALE_WH * jnp.log(g[2:] / db[2:])
    vec = jnp.concatenate([gxy, gwh], axis=0)
    d = p - vec
    ad = jnp.abs(d)
    sl1 = jnp.sum(jnp.where(ad < 1.0, 0.5 * d * d, ad - 0.5), axis=0)
    ll = jnp.where(posm, sl1, 0.0)

    @pl.when(j == 0)
    def _():
        locred_ref[...] = jnp.zeros_like(locred_ref)

    locred_ref[...] += jnp.sum(ll, axis=0, keepdims=True)


def _phase2_kernel(con_ref, glabel_ref, locred_ref, out_ref):
    lbl = glabel_ref[...]
    posm = lbl > 0
    con = con_ref[...]

    pos = jnp.sum(jnp.where(posm, 1.0, 0.0), axis=0, keepdims=True)
    conm = jnp.sum(jnp.where(posm, con, 0.0), axis=0, keepdims=True)
    locm = locred_ref[...]

    cn = jnp.where(posm, 0.0, con)
    ci = jax.lax.bitcast_convert_type(cn, jnp.int32)

    pos_i = pos.astype(jnp.int32)
    k = jnp.minimum(3 * pos_i, A)
    kk = jnp.maximum(k, 1).astype(jnp.float32)

    prefix = jnp.zeros((1, N), jnp.int32)
    krem = kk
    for b in range(30, -1, -1):
        hi_mask = jnp.int32(-(1 << b))
        cand = prefix | jnp.int32(1 << b)
        cnt = jnp.sum(jnp.where((ci & hi_mask) == cand, 1.0, 0.0),
                      axis=0, keepdims=True)
        take = krem <= cnt
        prefix = jnp.where(take, cand, prefix)
        krem = jnp.where(take, krem, krem - cnt)

    v = jax.lax.bitcast_convert_type(prefix, jnp.float32)
    gt = cn > v
    t_cnt = jnp.sum(jnp.where(gt, 1.0, 0.0), axis=0, keepdims=True)
    ns = jnp.sum(jnp.where(gt, cn, 0.0), axis=0, keepdims=True)
    neg_total = ns + (k.astype(jnp.float32) - t_cnt) * v

    total = locm + conm + neg_total
    contrib = jnp.where(pos > 0, total / jnp.maximum(pos, 1e-6), 0.0)
    out_ref[...] = jnp.sum(contrib, keepdims=True).reshape(1, 1) / N


@jax.jit
def kernel(ploc, plabel, gloc, glabel, dboxes):
    plabel_t = jnp.transpose(plabel, (1, 2, 0))
    ploc_t = jnp.transpose(ploc, (1, 2, 0))
    gloc_t = jnp.transpose(gloc, (1, 2, 0))
    glabel_t = glabel.T
    dbx = jnp.broadcast_to(dboxes[0][:, :, None], (4, A, N))

    con_t, locred = pl.pallas_call(
        _phase1_kernel,
        grid=(JA,),
        in_specs=[
            pl.BlockSpec((C, AB, N), lambda j: (0, j, 0)),
            pl.BlockSpec((AB, N), lambda j: (j, 0)),
            pl.BlockSpec((4, AB, N), lambda j: (0, j, 0)),
            pl.BlockSpec((4, AB, N), lambda j: (0, j, 0)),
            pl.BlockSpec((4, AB, N), lambda j: (0, j, 0)),
        ],
        out_specs=[
            pl.BlockSpec((AB, N), lambda j: (j, 0)),
            pl.BlockSpec((1, N), lambda j: (0, 0)),
        ],
        out_shape=[
            jax.ShapeDtypeStruct((A, N), jnp.float32),
            jax.ShapeDtypeStruct((1, N), jnp.float32),
        ],
    )(plabel_t, glabel_t, ploc_t, gloc_t, dbx)

    out = pl.pallas_call(
        _phase2_kernel,
        out_shape=jax.ShapeDtypeStruct((1, 1), jnp.float32),
    )(con_t, glabel_t, locred)
    return out[0, 0]


# speedup vs baseline: 3.8958x; 1.2581x over previous
import jax
import jax.numpy as jnp
from jax.experimental import pallas as pl
from jax.experimental.pallas import tpu as pltpu

N, A, C = 128, 8732, 81
SCALE_XY = 1.0 / 0.1
SCALE_WH = 1.0 / 0.2

AB = 256
JA = (A + AB - 1) // AB


def _phase1_kernel(plabel_ref, glabel_ref, ploc_ref, gloc_ref, dbx_ref,
                   cn_ref, stats_ref):
    j = pl.program_id(0)

    lbl = glabel_ref[...]
    arow = jax.lax.broadcasted_iota(jnp.int32, (AB, N), 0)
    valid = (j * AB + arow) < A
    posm = (lbl > 0) & valid

    x = plabel_ref[...]
    e = jnp.exp(x)
    s = jnp.sum(e, axis=0)
    logz = jnp.log(s)
    cidx = jax.lax.broadcasted_iota(jnp.int32, (C, AB, N), 0)
    tl = jnp.sum(jnp.where(cidx == lbl[None], x, 0.0), axis=0)
    con = logz - tl
    cn_ref[...] = jnp.where(posm, 0.0, con)

    p = ploc_ref[...]
    g = gloc_ref[...]
    db = dbx_ref[...]
    gxy = SCALE_XY * (g[:2] - db[:2]) / db[2:]
    gwh = SCALE_WH * jnp.log(g[2:] / db[2:])
    vec = jnp.concatenate([gxy, gwh], axis=0)
    d = p - vec
    ad = jnp.abs(d)
    sl1 = jnp.sum(jnp.where(ad < 1.0, 0.5 * d * d, ad - 0.5), axis=0)
    ll = jnp.where(posm, sl1, 0.0)

    upd = jnp.concatenate([
        jnp.sum(ll, axis=0, keepdims=True),
        jnp.sum(jnp.where(posm, 1.0, 0.0), axis=0, keepdims=True),
        jnp.sum(jnp.where(posm, con, 0.0), axis=0, keepdims=True),
        jnp.zeros((5, N), jnp.float32),
    ], axis=0)

    @pl.when(j == 0)
    def _():
        stats_ref[...] = jnp.zeros_like(stats_ref)

    stats_ref[...] += upd


def _phase2_kernel(cn_ref, stats_ref, out_ref, prefix_ref):
    st = stats_ref[...]
    locm = st[0:1]
    pos = st[1:2]
    conm = st[2:3]

    cn = cn_ref[...]
    ci = jax.lax.bitcast_convert_type(cn, jnp.int32)

    pos_i = pos.astype(jnp.int32)
    k = jnp.minimum(3 * pos_i, A)
    kk = jnp.maximum(k, 1).astype(jnp.float32)

    prefix_ref[...] = jnp.zeros((1, N), jnp.int32)
    need_select = jnp.min(3 * pos_i) < A

    @pl.when(need_select)
    def _():
        prefix = jnp.zeros((1, N), jnp.int32)
        krem = kk
        for b in range(30, -1, -1):
            hi_mask = jnp.int32(-(1 << b))
            cand = prefix | jnp.int32(1 << b)
            cnt = jnp.sum(jnp.where((ci & hi_mask) == cand, 1.0, 0.0),
                          axis=0, keepdims=True)
            take = krem <= cnt
            prefix = jnp.where(take, cand, prefix)
            krem = jnp.where(take, krem, krem - cnt)
        prefix_ref[...] = prefix

    v = jax.lax.bitcast_convert_type(prefix_ref[...], jnp.float32)
    gt = cn > v
    t_cnt = jnp.sum(jnp.where(gt, 1.0, 0.0), axis=0, keepdims=True)
    ns = jnp.sum(jnp.where(gt, cn, 0.0), axis=0, keepdims=True)
    neg_total = ns + (k.astype(jnp.float32) - t_cnt) * v

    total = locm + conm + neg_total
    contrib = jnp.where(pos > 0, total / jnp.maximum(pos, 1e-6), 0.0)
    out_ref[...] = jnp.sum(contrib, keepdims=True).reshape(1, 1) / N


@jax.jit
def kernel(ploc, plabel, gloc, glabel, dboxes):
    plabel_t = jnp.transpose(plabel, (1, 2, 0))
    ploc_t = jnp.transpose(ploc, (1, 2, 0))
    gloc_t = jnp.transpose(gloc, (1, 2, 0))
    glabel_t = glabel.T
    dbx = jnp.broadcast_to(dboxes[0][:, :, None], (4, A, N))

    cn, stats = pl.pallas_call(
        _phase1_kernel,
        grid=(JA,),
        in_specs=[
            pl.BlockSpec((C, AB, N), lambda j: (0, j, 0)),
            pl.BlockSpec((AB, N), lambda j: (j, 0)),
            pl.BlockSpec((4, AB, N), lambda j: (0, j, 0)),
            pl.BlockSpec((4, AB, N), lambda j: (0, j, 0)),
            pl.BlockSpec((4, AB, N), lambda j: (0, j, 0)),
        ],
        out_specs=[
            pl.BlockSpec((AB, N), lambda j: (j, 0)),
            pl.BlockSpec((8, N), lambda j: (0, 0)),
        ],
        out_shape=[
            jax.ShapeDtypeStruct((A, N), jnp.float32),
            jax.ShapeDtypeStruct((8, N), jnp.float32),
        ],
    )(plabel_t, glabel_t, ploc_t, gloc_t, dbx)

    out = pl.pallas_call(
        _phase2_kernel,
        out_shape=jax.ShapeDtypeStruct((1, 1), jnp.float32),
        scratch_shapes=[pltpu.VMEM((1, N), jnp.int32)],
    )(cn, stats)
    return out[0, 0]
